# baseline (speedup 1.0000x reference)
import jax
import jax.numpy as jnp
from jax.experimental import pallas as pl
from jax.experimental.pallas import tpu as pltpu

LANE = 128
ROWS_PER_STEP = 8
_VMEM_LIMIT = 64 * 1024 * 1024

_TN = (((0,), (0,)), ((), ()))


def _fused_kernel(x_ref, e_ref, table_ref, w1b_ref, b1b_ref, w2_ref, b2_ref,
                  node_out_ref, edge_out_ref):
    r = x_ref.shape[0]
    x = x_ref[...]
    e = e_ref[...]
    srow = jax.lax.broadcasted_iota(jnp.int32, (LANE, LANE), 0)

    oh_chunks = []
    h_chunks = []
    for i in range(r):
        xi = jnp.broadcast_to(x[i:i + 1, :], (LANE, LANE))
        oh_chunks.append((xi == srow).astype(jnp.bfloat16))
        ei = jnp.broadcast_to(e[i:i + 1, :], (LANE, LANE))
        h = ei * w1b_ref[...] + b1b_ref[...]
        h_chunks.append((h * (0.5 + 0.5 * jnp.tanh(0.5 * h))).astype(jnp.bfloat16))

    onehot_t = jnp.concatenate(oh_chunks, axis=1)
    node_out_ref[...] = jax.lax.dot_general(
        onehot_t, table_ref[...], _TN,
        preferred_element_type=jnp.float32,
    )

    h_t = jnp.concatenate(h_chunks, axis=1)
    edge_out_ref[...] = jax.lax.dot_general(
        h_t, w2_ref[...], _TN,
        preferred_element_type=jnp.float32,
    ) + b2_ref[...]


def kernel(embed_node, edge_w1, edge_b1, edge_w2, edge_b2, x, edge_attr):
    n = x.shape[0]
    s, d = embed_node.shape
    assert edge_attr.shape[0] == n and edge_attr.shape[1] == 1
    assert d == LANE and edge_w2.shape == (d, d)
    assert n % (ROWS_PER_STEP * LANE) == 0 and s <= LANE

    n_rows = n // LANE
    tile_rows = n // (ROWS_PER_STEP * LANE)
    tile = ROWS_PER_STEP * LANE

    x2 = x.astype(jnp.int32).reshape(n_rows, LANE)
    e2 = edge_attr.reshape(n_rows, LANE)

    table_p = jnp.pad(embed_node, ((0, LANE - s), (0, 0))).astype(jnp.bfloat16)
    w2b = edge_w2.astype(jnp.bfloat16)
    w1b = jnp.broadcast_to(edge_w1.reshape(d, 1), (d, LANE))
    b1b = jnp.broadcast_to(edge_b1.reshape(d, 1), (d, LANE))

    h_node, h_edge = pl.pallas_call(
        _fused_kernel,
        out_shape=(
            jax.ShapeDtypeStruct((n, d), jnp.float32),
            jax.ShapeDtypeStruct((n, d), jnp.float32),
        ),
        grid=(tile_rows,),
        in_specs=[
            pl.BlockSpec((ROWS_PER_STEP, LANE), lambda i: (i, 0)),
            pl.BlockSpec((ROWS_PER_STEP, LANE), lambda i: (i, 0)),
            pl.BlockSpec((LANE, d), lambda i: (0, 0)),
            pl.BlockSpec((d, LANE), lambda i: (0, 0)),
            pl.BlockSpec((d, LANE), lambda i: (0, 0)),
            pl.BlockSpec((d, d), lambda i: (0, 0)),
            pl.BlockSpec((1, d), lambda i: (0, 0)),
        ],
        out_specs=(
            pl.BlockSpec((tile, d), lambda i: (i, 0)),
            pl.BlockSpec((tile, d), lambda i: (i, 0)),
        ),
        compiler_params=pltpu.CompilerParams(
            dimension_semantics=("parallel",),
            vmem_limit_bytes=_VMEM_LIMIT,
        ),
    )(x2, e2, table_p, w1b, b1b, w2b, edge_b2)
    return h_node, h_edge


# speedup vs baseline: 2.0819x; 2.0819x over previous
import jax
import jax.numpy as jnp
from jax.experimental import pallas as pl
from jax.experimental.pallas import tpu as pltpu

LANE = 128
ROWS_PER_STEP = 32
_VMEM_LIMIT = 64 * 1024 * 1024

_TN = (((0,), (0,)), ((), ()))


def _fused_kernel(x_ref, e_ref, table_ref, w1b_ref, b1b_ref, w2_ref, b2_ref,
                  node_out_ref, edge_out_ref):
    r = x_ref.shape[0]
    x = x_ref[...]
    e = e_ref[...]
    srow = jax.lax.broadcasted_iota(jnp.int32, (LANE, LANE), 0)

    oh_chunks = []
    h_chunks = []
    for i in range(r):
        xi = jnp.broadcast_to(x[i:i + 1, :], (LANE, LANE))
        oh_chunks.append((xi == srow).astype(jnp.bfloat16))
        ei = jnp.broadcast_to(e[i:i + 1, :], (LANE, LANE))
        h = ei * w1b_ref[...] + b1b_ref[...]
        h_chunks.append((h * (0.5 + 0.5 * jnp.tanh(0.5 * h))).astype(jnp.bfloat16))

    onehot_t = jnp.concatenate(oh_chunks, axis=1)
    node_out_ref[...] = jax.lax.dot_general(
        onehot_t, table_ref[...], _TN,
        preferred_element_type=jnp.float32,
    )

    h_t = jnp.concatenate(h_chunks, axis=1)
    edge_out_ref[...] = jax.lax.dot_general(
        h_t, w2_ref[...], _TN,
        preferred_element_type=jnp.float32,
    ) + b2_ref[...]


def kernel(embed_node, edge_w1, edge_b1, edge_w2, edge_b2, x, edge_attr):
    n = x.shape[0]
    s, d = embed_node.shape
    assert edge_attr.shape[0] == n and edge_attr.shape[1] == 1
    assert d == LANE and edge_w2.shape == (d, d)
    assert n % (ROWS_PER_STEP * LANE) == 0 and s <= LANE

    n_rows = n // LANE
    tile_rows = n // (ROWS_PER_STEP * LANE)
    tile = ROWS_PER_STEP * LANE

    x2 = x.astype(jnp.int32).reshape(n_rows, LANE)
    e2 = edge_attr.reshape(n_rows, LANE)

    table_p = jnp.pad(embed_node, ((0, LANE - s), (0, 0))).astype(jnp.bfloat16)
    w2b = edge_w2.astype(jnp.bfloat16)
    w1b = jnp.broadcast_to(edge_w1.reshape(d, 1), (d, LANE))
    b1b = jnp.broadcast_to(edge_b1.reshape(d, 1), (d, LANE))

    h_node, h_edge = pl.pallas_call(
        _fused_kernel,
        out_shape=(
            jax.ShapeDtypeStruct((n, d), jnp.float32),
            jax.ShapeDtypeStruct((n, d), jnp.float32),
        ),
        grid=(tile_rows,),
        in_specs=[
            pl.BlockSpec((ROWS_PER_STEP, LANE), lambda i: (i, 0)),
            pl.BlockSpec((ROWS_PER_STEP, LANE), lambda i: (i, 0)),
            pl.BlockSpec((LANE, d), lambda i: (0, 0)),
            pl.BlockSpec((d, LANE), lambda i: (0, 0)),
            pl.BlockSpec((d, LANE), lambda i: (0, 0)),
            pl.BlockSpec((d, d), lambda i: (0, 0)),
            pl.BlockSpec((1, d), lambda i: (0, 0)),
        ],
        out_specs=(
            pl.BlockSpec((tile, d), lambda i: (i, 0)),
            pl.BlockSpec((tile, d), lambda i: (i, 0)),
        ),
        compiler_params=pltpu.CompilerParams(
            dimension_semantics=("parallel",),
            vmem_limit_bytes=_VMEM_LIMIT,
        ),
    )(x2, e2, table_p, w1b, b1b, w2b, edge_b2)
    return h_node, h_edge


# speedup vs baseline: 2.4511x; 1.1773x over previous
import jax
import jax.numpy as jnp
from jax.experimental import pallas as pl
from jax.experimental.pallas import tpu as pltpu

LANE = 128
ROWS_PER_STEP = 64
_VMEM_LIMIT = 64 * 1024 * 1024

_TN = (((0,), (0,)), ((), ()))


def _fused_kernel(x_ref, e_ref, table_ref, w1b_ref, b1b_ref, w2_ref, b2_ref,
                  node_out_ref, edge_out_ref):
    r = x_ref.shape[0]
    x = x_ref[...]
    e = e_ref[...]
    srow = jax.lax.broadcasted_iota(jnp.int32, (LANE, LANE), 0)

    oh_chunks = []
    h_chunks = []
    for i in range(r):
        xi = jnp.broadcast_to(x[i:i + 1, :], (LANE, LANE))
        oh_chunks.append((xi == srow).astype(jnp.bfloat16))
        ei = jnp.broadcast_to(e[i:i + 1, :], (LANE, LANE))
        h = ei * w1b_ref[...] + b1b_ref[...]
        h_chunks.append((h * (0.5 + 0.5 * jnp.tanh(0.5 * h))).astype(jnp.bfloat16))

    onehot_t = jnp.concatenate(oh_chunks, axis=1)
    node_out_ref[...] = jax.lax.dot_general(
        onehot_t, table_ref[...], _TN,
        preferred_element_type=jnp.float32,
    )

    h_t = jnp.concatenate(h_chunks, axis=1)
    edge_out_ref[...] = jax.lax.dot_general(
        h_t, w2_ref[...], _TN,
        preferred_element_type=jnp.float32,
    ) + b2_ref[...]


def kernel(embed_node, edge_w1, edge_b1, edge_w2, edge_b2, x, edge_attr):
    n = x.shape[0]
    s, d = embed_node.shape
    assert edge_attr.shape[0] == n and edge_attr.shape[1] == 1
    assert d == LANE and edge_w2.shape == (d, d)
    assert n % (ROWS_PER_STEP * LANE) == 0 and s <= LANE

    n_rows = n // LANE
    tile_rows = n // (ROWS_PER_STEP * LANE)
    tile = ROWS_PER_STEP * LANE

    x2 = x.astype(jnp.int32).reshape(n_rows, LANE)
    e2 = edge_attr.reshape(n_rows, LANE)

    table_p = jnp.pad(embed_node, ((0, LANE - s), (0, 0))).astype(jnp.bfloat16)
    w2b = edge_w2.astype(jnp.bfloat16)
    w1b = jnp.broadcast_to(edge_w1.reshape(d, 1), (d, LANE))
    b1b = jnp.broadcast_to(edge_b1.reshape(d, 1), (d, LANE))

    h_node, h_edge = pl.pallas_call(
        _fused_kernel,
        out_shape=(
            jax.ShapeDtypeStruct((n, d), jnp.float32),
            jax.ShapeDtypeStruct((n, d), jnp.float32),
        ),
        grid=(tile_rows,),
        in_specs=[
            pl.BlockSpec((ROWS_PER_STEP, LANE), lambda i: (i, 0)),
            pl.BlockSpec((ROWS_PER_STEP, LANE), lambda i: (i, 0)),
            pl.BlockSpec((LANE, d), lambda i: (0, 0)),
            pl.BlockSpec((d, LANE), lambda i: (0, 0)),
            pl.BlockSpec((d, LANE), lambda i: (0, 0)),
            pl.BlockSpec((d, d), lambda i: (0, 0)),
            pl.BlockSpec((1, d), lambda i: (0, 0)),
        ],
        out_specs=(
            pl.BlockSpec((tile, d), lambda i: (i, 0)),
            pl.BlockSpec((tile, d), lambda i: (i, 0)),
        ),
        compiler_params=pltpu.CompilerParams(
            dimension_semantics=("parallel",),
            vmem_limit_bytes=_VMEM_LIMIT,
        ),
    )(x2, e2, table_p, w1b, b1b, w2b, edge_b2)
    return h_node, h_edge


# speedup vs baseline: 2.4644x; 1.0054x over previous
import jax
import jax.numpy as jnp
from jax.experimental import pallas as pl
from jax.experimental.pallas import tpu as pltpu

LANE = 128
ROWS_PER_STEP = 128
_VMEM_LIMIT = 64 * 1024 * 1024

_TN = (((0,), (0,)), ((), ()))


def _fused_kernel(x_ref, e_ref, table_ref, w1b_ref, b1b_ref, w2_ref, b2_ref,
                  node_out_ref, edge_out_ref):
    r = x_ref.shape[0]
    x = x_ref[...]
    e = e_ref[...]
    srow = jax.lax.broadcasted_iota(jnp.int32, (LANE, LANE), 0)

    oh_chunks = []
    h_chunks = []
    for i in range(r):
        xi = jnp.broadcast_to(x[i:i + 1, :], (LANE, LANE))
        oh_chunks.append((xi == srow).astype(jnp.bfloat16))
        ei = jnp.broadcast_to(e[i:i + 1, :], (LANE, LANE))
        h = ei * w1b_ref[...] + b1b_ref[...]
        h_chunks.append((h * (0.5 + 0.5 * jnp.tanh(0.5 * h))).astype(jnp.bfloat16))

    onehot_t = jnp.concatenate(oh_chunks, axis=1)
    node_out_ref[...] = jax.lax.dot_general(
        onehot_t, table_ref[...], _TN,
        preferred_element_type=jnp.float32,
    )

    h_t = jnp.concatenate(h_chunks, axis=1)
    edge_out_ref[...] = jax.lax.dot_general(
        h_t, w2_ref[...], _TN,
        preferred_element_type=jnp.float32,
    ) + b2_ref[...]


def kernel(embed_node, edge_w1, edge_b1, edge_w2, edge_b2, x, edge_attr):
    n = x.shape[0]
    s, d = embed_node.shape
    assert edge_attr.shape[0] == n and edge_attr.shape[1] == 1
    assert d == LANE and edge_w2.shape == (d, d)
    assert n % (ROWS_PER_STEP * LANE) == 0 and s <= LANE

    n_rows = n // LANE
    tile_rows = n // (ROWS_PER_STEP * LANE)
    tile = ROWS_PER_STEP * LANE

    x2 = x.astype(jnp.int32).reshape(n_rows, LANE)
    e2 = edge_attr.reshape(n_rows, LANE)

    table_p = jnp.pad(embed_node, ((0, LANE - s), (0, 0))).astype(jnp.bfloat16)
    w2b = edge_w2.astype(jnp.bfloat16)
    w1b = jnp.broadcast_to(edge_w1.reshape(d, 1), (d, LANE))
    b1b = jnp.broadcast_to(edge_b1.reshape(d, 1), (d, LANE))

    h_node, h_edge = pl.pallas_call(
        _fused_kernel,
        out_shape=(
            jax.ShapeDtypeStruct((n, d), jnp.float32),
            jax.ShapeDtypeStruct((n, d), jnp.float32),
        ),
        grid=(tile_rows,),
        in_specs=[
            pl.BlockSpec((ROWS_PER_STEP, LANE), lambda i: (i, 0)),
            pl.BlockSpec((ROWS_PER_STEP, LANE), lambda i: (i, 0)),
            pl.BlockSpec((LANE, d), lambda i: (0, 0)),
            pl.BlockSpec((d, LANE), lambda i: (0, 0)),
            pl.BlockSpec((d, LANE), lambda i: (0, 0)),
            pl.BlockSpec((d, d), lambda i: (0, 0)),
            pl.BlockSpec((1, d), lambda i: (0, 0)),
        ],
        out_specs=(
            pl.BlockSpec((tile, d), lambda i: (i, 0)),
            pl.BlockSpec((tile, d), lambda i: (i, 0)),
        ),
        compiler_params=pltpu.CompilerParams(
            dimension_semantics=("parallel",),
            vmem_limit_bytes=_VMEM_LIMIT,
        ),
    )(x2, e2, table_p, w1b, b1b, w2b, edge_b2)
    return h_node, h_edge
